# baseline (speedup 1.0000x reference)
import jax
import jax.numpy as jnp
from jax.experimental import pallas as pl
from jax.experimental.pallas import tpu as pltpu

K_NB = 16
RIDGE = 1e-06
BM = 256
_INF = float("inf")
_HI = jax.lax.Precision.HIGHEST


def _knn_div2_kernel(qt_ref, r_ref, rt_ref, ut_ref, out_ref, mask_ref):
    qt = qt_ref[0]
    r = r_ref[0]
    rt = rt_ref[0]
    ut = ut_ref[0]
    n = r.shape[0]

    qn = jnp.sum(qt * qt, axis=0, keepdims=True)
    rn = jnp.sum(r * r, axis=1, keepdims=True)
    qr = jax.lax.dot_general(r.astype(jnp.bfloat16), qt.astype(jnp.bfloat16),
                             (((1,), (0,)), ((), ())),
                             preferred_element_type=jnp.float32)
    d0 = (qn + rn) - 2.0 * qr
    d0 = jnp.sqrt(jnp.maximum(d0, 0.0))

    iota = jax.lax.broadcasted_iota(jnp.int32, (n, BM), 0)
    nbig = jnp.int32(n)

    m0 = jnp.min(d0, axis=0, keepdims=True)
    g0 = d0 == m0
    idx0 = jnp.min(jnp.where(g0, iota, nbig), axis=0, keepdims=True)
    d = jnp.where(g0, _INF, d0)
    for _ in range(K_NB - 1):
        m = jnp.min(d, axis=0, keepdims=True)
        d = jnp.where(d == m, _INF, d)
    mask_ref[...] = (d == _INF).astype(jnp.float32)

    cnt = jnp.sum(mask_ref[...], axis=0, keepdims=True)
    overshoot = jnp.max(cnt) > jnp.float32(K_NB)

    @pl.when(overshoot)
    def _fixup():
        dd = jnp.where(mask_ref[...] > 0.0, d0, _INF)
        msk = jnp.zeros((n, BM), jnp.float32)
        for _ in range(K_NB):
            mm = jnp.min(dd, axis=0, keepdims=True)
            ii = jnp.min(jnp.where(dd == mm, iota, nbig), axis=0,
                         keepdims=True)
            oh = iota == ii
            msk = msk + oh.astype(jnp.float32)
            dd = jnp.where(oh, _INF, dd)
        mask_ref[...] = msk

    maskt = mask_ref[...]
    oh0t = (iota == idx0).astype(jnp.float32)

    x = rt[0:1, :]
    y = rt[1:2, :]
    z = rt[2:3, :]
    featt = jnp.concatenate(
        [rt,
         x * rt,
         y * rt[1:3, :],
         z * z,
         ut,
         x * ut,
         y * ut,
         z * ut],
        axis=0)
    S = jax.lax.dot_general(featt, maskt, (((1,), (0,)), ((), ())),
                            preferred_element_type=jnp.float32,
                            precision=_HI)
    fq = jax.lax.dot_general(ut, oh0t, (((1,), (0,)), ((), ())),
                             preferred_element_type=jnp.float32,
                             precision=_HI)

    kf = jnp.float32(K_NB)
    qa = [qt[a:a + 1, :] for a in range(3)]
    S1 = [S[a:a + 1, :] for a in range(3)]
    Sxx, Sxy, Sxz = S[3:4, :], S[4:5, :], S[5:6, :]
    Syy, Syz, Szz = S[6:7, :], S[7:8, :], S[8:9, :]
    Sf = [S[9 + c:10 + c, :] for c in range(3)]
    Sfr = [[S[12 + 3 * a + c:13 + 3 * a + c, :] for c in range(3)]
           for a in range(3)]
    fqc = [fq[c:c + 1, :] for c in range(3)]

    def xtx(s2, a, b):
        return s2 - qa[a] * S1[b] - qa[b] * S1[a] + kf * qa[a] * qa[b]

    A00 = xtx(Sxx, 0, 0)
    A01 = xtx(Sxy, 0, 1)
    A02 = xtx(Sxz, 0, 2)
    A11 = xtx(Syy, 1, 1)
    A12 = xtx(Syz, 1, 2)
    A22 = xtx(Szz, 2, 2)

    base = (jnp.abs(A00) + jnp.abs(A11) + jnp.abs(A22)
            + 2.0 * (jnp.abs(A01) + jnp.abs(A02) + jnp.abs(A12))) / 9.0 + 1e-12
    rb = jnp.float32(RIDGE) * base
    A00 = A00 + rb
    A11 = A11 + rb
    A22 = A22 + rb

    def xty(a, c):
        return (Sfr[a][c] - qa[a] * Sf[c] - fqc[c] * S1[a]
                + kf * qa[a] * fqc[c])

    Y = [[xty(a, c) for c in range(3)] for a in range(3)]

    adj00 = A11 * A22 - A12 * A12
    adj01 = A02 * A12 - A01 * A22
    adj02 = A01 * A12 - A11 * A02
    adj11 = A00 * A22 - A02 * A02
    adj12 = A01 * A02 - A00 * A12
    adj22 = A00 * A11 - A01 * A01
    det = A00 * adj00 + A01 * adj01 + A02 * adj02

    num = (adj00 * Y[0][0] + adj01 * Y[1][0] + adj02 * Y[2][0]
           + adj01 * Y[0][1] + adj11 * Y[1][1] + adj12 * Y[2][1]
           + adj02 * Y[0][2] + adj12 * Y[1][2] + adj22 * Y[2][2])
    div = num / det
    out_ref[0, 0] = div * div


def _build_call(B, M, N, interpret=False):
    nb = M // BM
    grid = (B, nb)
    return pl.pallas_call(
        _knn_div2_kernel,
        grid=grid,
        in_specs=[
            pl.BlockSpec((1, 3, BM), lambda b, j: (b, 0, j)),
            pl.BlockSpec((1, N, 3), lambda b, j: (b, 0, 0)),
            pl.BlockSpec((1, 3, N), lambda b, j: (b, 0, 0)),
            pl.BlockSpec((1, 3, N), lambda b, j: (b, 0, 0)),
        ],
        out_specs=pl.BlockSpec((1, 1, 1, BM), lambda b, j: (b, j, 0, 0)),
        out_shape=jax.ShapeDtypeStruct((B, nb, 1, BM), jnp.float32),
        scratch_shapes=[pltpu.VMEM((N, BM), jnp.float32)],
        compiler_params=pltpu.CompilerParams(
            dimension_semantics=("parallel", "parallel")),
        interpret=interpret,
    )


def kernel(query_xyz, ref_xyz, u):
    B, M, _ = query_xyz.shape
    N = ref_xyz.shape[1]
    qt = jnp.swapaxes(query_xyz, 1, 2)
    rt = jnp.swapaxes(ref_xyz, 1, 2)
    ut = jnp.swapaxes(u, 1, 2)
    div2 = _build_call(B, M, N)(qt, ref_xyz, rt, ut)
    return jnp.mean(div2)


# speedup vs baseline: 1.3429x; 1.3429x over previous
import jax
import jax.numpy as jnp
from jax.experimental import pallas as pl
from jax.experimental.pallas import tpu as pltpu

K_NB = 16
RIDGE = 1e-06
BM = 256
_INF = float("inf")
_HI = jax.lax.Precision.HIGHEST


def _half_div2(qt, r, rbf, rn, featt, ut, n):
    bq = qt.shape[1]

    qn = jnp.sum(qt * qt, axis=0, keepdims=True)
    qr = jax.lax.dot_general(rbf, qt.astype(jnp.bfloat16),
                             (((1,), (0,)), ((), ())),
                             preferred_element_type=jnp.float32)
    d = (qn + rn) - 2.0 * qr
    d = jnp.sqrt(jnp.maximum(d, 0.0))

    iota = jax.lax.broadcasted_iota(jnp.int32, (n, bq), 0)
    nbig = jnp.int32(n)
    idx0 = None
    for j in range(K_NB):
        m = jnp.min(d, axis=0, keepdims=True)
        idx = jnp.min(jnp.where(d == m, iota, nbig), axis=0, keepdims=True)
        if j == 0:
            idx0 = idx
        d = jnp.where(iota == idx, _INF, d)
    maskt = (d == _INF).astype(jnp.float32)
    oh0t = (iota == idx0).astype(jnp.float32)

    S = jax.lax.dot_general(featt, maskt, (((1,), (0,)), ((), ())),
                            preferred_element_type=jnp.float32,
                            precision=_HI)
    fq = jax.lax.dot_general(ut, oh0t, (((1,), (0,)), ((), ())),
                             preferred_element_type=jnp.float32,
                             precision=_HI)

    kf = jnp.float32(K_NB)
    qa = [qt[a:a + 1, :] for a in range(3)]
    S1 = [S[a:a + 1, :] for a in range(3)]
    Sxx, Sxy, Sxz = S[3:4, :], S[4:5, :], S[5:6, :]
    Syy, Syz, Szz = S[6:7, :], S[7:8, :], S[8:9, :]
    Sf = [S[9 + c:10 + c, :] for c in range(3)]
    Sfr = [[S[12 + 3 * a + c:13 + 3 * a + c, :] for c in range(3)]
           for a in range(3)]
    fqc = [fq[c:c + 1, :] for c in range(3)]

    def xtx(s2, a, b):
        return s2 - qa[a] * S1[b] - qa[b] * S1[a] + kf * qa[a] * qa[b]

    A00 = xtx(Sxx, 0, 0)
    A01 = xtx(Sxy, 0, 1)
    A02 = xtx(Sxz, 0, 2)
    A11 = xtx(Syy, 1, 1)
    A12 = xtx(Syz, 1, 2)
    A22 = xtx(Szz, 2, 2)

    base = (jnp.abs(A00) + jnp.abs(A11) + jnp.abs(A22)
            + 2.0 * (jnp.abs(A01) + jnp.abs(A02) + jnp.abs(A12))) / 9.0 + 1e-12
    rb = jnp.float32(RIDGE) * base
    A00 = A00 + rb
    A11 = A11 + rb
    A22 = A22 + rb

    def xty(a, c):
        return (Sfr[a][c] - qa[a] * Sf[c] - fqc[c] * S1[a]
                + kf * qa[a] * fqc[c])

    Y = [[xty(a, c) for c in range(3)] for a in range(3)]

    adj00 = A11 * A22 - A12 * A12
    adj01 = A02 * A12 - A01 * A22
    adj02 = A01 * A12 - A11 * A02
    adj11 = A00 * A22 - A02 * A02
    adj12 = A01 * A02 - A00 * A12
    adj22 = A00 * A11 - A01 * A01
    det = A00 * adj00 + A01 * adj01 + A02 * adj02

    num = (adj00 * Y[0][0] + adj01 * Y[1][0] + adj02 * Y[2][0]
           + adj01 * Y[0][1] + adj11 * Y[1][1] + adj12 * Y[2][1]
           + adj02 * Y[0][2] + adj12 * Y[1][2] + adj22 * Y[2][2])
    div = num / det
    return div * div


def _knn_div2_kernel(qt_ref, r_ref, rt_ref, ut_ref, out_ref):
    qt = qt_ref[0]
    r = r_ref[0]
    rt = rt_ref[0]
    ut = ut_ref[0]
    n = r.shape[0]

    rbf = r.astype(jnp.bfloat16)
    rn = jnp.sum(r * r, axis=1, keepdims=True)

    x = rt[0:1, :]
    y = rt[1:2, :]
    z = rt[2:3, :]
    featt = jnp.concatenate(
        [rt,
         x * rt,
         y * rt[1:3, :],
         z * z,
         ut,
         x * ut,
         y * ut,
         z * ut],
        axis=0)

    h = BM // 2
    d0 = _half_div2(qt[:, 0:h], r, rbf, rn, featt, ut, n)
    d1 = _half_div2(qt[:, h:BM], r, rbf, rn, featt, ut, n)
    out_ref[0, 0] = jnp.concatenate([d0, d1], axis=1)


def _build_call(B, M, N, interpret=False):
    nb = M // BM
    grid = (B, nb)
    return pl.pallas_call(
        _knn_div2_kernel,
        grid=grid,
        in_specs=[
            pl.BlockSpec((1, 3, BM), lambda b, j: (b, 0, j)),
            pl.BlockSpec((1, N, 3), lambda b, j: (b, 0, 0)),
            pl.BlockSpec((1, 3, N), lambda b, j: (b, 0, 0)),
            pl.BlockSpec((1, 3, N), lambda b, j: (b, 0, 0)),
        ],
        out_specs=pl.BlockSpec((1, 1, 1, BM), lambda b, j: (b, j, 0, 0)),
        out_shape=jax.ShapeDtypeStruct((B, nb, 1, BM), jnp.float32),
        compiler_params=pltpu.CompilerParams(
            dimension_semantics=("parallel", "parallel")),
        interpret=interpret,
    )


def kernel(query_xyz, ref_xyz, u):
    B, M, _ = query_xyz.shape
    N = ref_xyz.shape[1]
    qt = jnp.swapaxes(query_xyz, 1, 2)
    rt = jnp.swapaxes(ref_xyz, 1, 2)
    ut = jnp.swapaxes(u, 1, 2)
    div2 = _build_call(B, M, N)(qt, ref_xyz, rt, ut)
    return jnp.mean(div2)


# speedup vs baseline: 2.0618x; 1.5353x over previous
import jax
import jax.numpy as jnp
from jax.experimental import pallas as pl
from jax.experimental.pallas import tpu as pltpu

K_NB = 16
RIDGE = 1e-06
BM = 512
_INF = float("inf")
_HI = jax.lax.Precision.HIGHEST


def _knn_div2_kernel(qt_ref, r_ref, rt_ref, ut_ref, out_ref):
    qt = qt_ref[0]
    r = r_ref[0]
    rt = rt_ref[0]
    ut = ut_ref[0]
    n = r.shape[0]

    qn = jnp.sum(qt * qt, axis=0, keepdims=True)
    rn = jnp.sum(r * r, axis=1, keepdims=True)
    qr = jax.lax.dot_general(r.astype(jnp.bfloat16), qt.astype(jnp.bfloat16),
                             (((1,), (0,)), ((), ())),
                             preferred_element_type=jnp.float32)
    d = (qn + rn) - 2.0 * qr
    d = jnp.sqrt(jnp.maximum(d, 0.0))

    iota = jax.lax.broadcasted_iota(jnp.int32, (n, BM), 0)
    nbig = jnp.int32(n)
    idx0 = None
    for j in range(K_NB):
        m = jnp.min(d, axis=0, keepdims=True)
        idx = jnp.min(jnp.where(d == m, iota, nbig), axis=0, keepdims=True)
        if j == 0:
            idx0 = idx
        d = jnp.where(iota == idx, _INF, d)
    maskt = (d == _INF).astype(jnp.float32)
    oh0t = (iota == idx0).astype(jnp.float32)

    x = rt[0:1, :]
    y = rt[1:2, :]
    z = rt[2:3, :]
    featt = jnp.concatenate(
        [rt,
         x * rt,
         y * rt[1:3, :],
         z * z,
         ut,
         x * ut,
         y * ut,
         z * ut],
        axis=0)
    S = jax.lax.dot_general(featt, maskt, (((1,), (0,)), ((), ())),
                            preferred_element_type=jnp.float32,
                            precision=_HI)
    fq = jax.lax.dot_general(ut, oh0t, (((1,), (0,)), ((), ())),
                             preferred_element_type=jnp.float32,
                             precision=_HI)

    kf = jnp.float32(K_NB)
    qa = [qt[a:a + 1, :] for a in range(3)]
    S1 = [S[a:a + 1, :] for a in range(3)]
    Sxx, Sxy, Sxz = S[3:4, :], S[4:5, :], S[5:6, :]
    Syy, Syz, Szz = S[6:7, :], S[7:8, :], S[8:9, :]
    Sf = [S[9 + c:10 + c, :] for c in range(3)]
    Sfr = [[S[12 + 3 * a + c:13 + 3 * a + c, :] for c in range(3)]
           for a in range(3)]
    fqc = [fq[c:c + 1, :] for c in range(3)]

    def xtx(s2, a, b):
        return s2 - qa[a] * S1[b] - qa[b] * S1[a] + kf * qa[a] * qa[b]

    A00 = xtx(Sxx, 0, 0)
    A01 = xtx(Sxy, 0, 1)
    A02 = xtx(Sxz, 0, 2)
    A11 = xtx(Syy, 1, 1)
    A12 = xtx(Syz, 1, 2)
    A22 = xtx(Szz, 2, 2)

    base = (jnp.abs(A00) + jnp.abs(A11) + jnp.abs(A22)
            + 2.0 * (jnp.abs(A01) + jnp.abs(A02) + jnp.abs(A12))) / 9.0 + 1e-12
    rb = jnp.float32(RIDGE) * base
    A00 = A00 + rb
    A11 = A11 + rb
    A22 = A22 + rb

    def xty(a, c):
        return (Sfr[a][c] - qa[a] * Sf[c] - fqc[c] * S1[a]
                + kf * qa[a] * fqc[c])

    Y = [[xty(a, c) for c in range(3)] for a in range(3)]

    adj00 = A11 * A22 - A12 * A12
    adj01 = A02 * A12 - A01 * A22
    adj02 = A01 * A12 - A11 * A02
    adj11 = A00 * A22 - A02 * A02
    adj12 = A01 * A02 - A00 * A12
    adj22 = A00 * A11 - A01 * A01
    det = A00 * adj00 + A01 * adj01 + A02 * adj02

    num = (adj00 * Y[0][0] + adj01 * Y[1][0] + adj02 * Y[2][0]
           + adj01 * Y[0][1] + adj11 * Y[1][1] + adj12 * Y[2][1]
           + adj02 * Y[0][2] + adj12 * Y[1][2] + adj22 * Y[2][2])
    div = num / det
    out_ref[0, 0] = div * div


def _build_call(B, M, N, interpret=False):
    nb = M // BM
    grid = (B, nb)
    return pl.pallas_call(
        _knn_div2_kernel,
        grid=grid,
        in_specs=[
            pl.BlockSpec((1, 3, BM), lambda b, j: (b, 0, j)),
            pl.BlockSpec((1, N, 3), lambda b, j: (b, 0, 0)),
            pl.BlockSpec((1, 3, N), lambda b, j: (b, 0, 0)),
            pl.BlockSpec((1, 3, N), lambda b, j: (b, 0, 0)),
        ],
        out_specs=pl.BlockSpec((1, 1, 1, BM), lambda b, j: (b, j, 0, 0)),
        out_shape=jax.ShapeDtypeStruct((B, nb, 1, BM), jnp.float32),
        compiler_params=pltpu.CompilerParams(
            dimension_semantics=("parallel", "parallel")),
        interpret=interpret,
    )


def kernel(query_xyz, ref_xyz, u):
    B, M, _ = query_xyz.shape
    N = ref_xyz.shape[1]
    qt = jnp.swapaxes(query_xyz, 1, 2)
    rt = jnp.swapaxes(ref_xyz, 1, 2)
    ut = jnp.swapaxes(u, 1, 2)
    div2 = _build_call(B, M, N)(qt, ref_xyz, rt, ut)
    return jnp.mean(div2)
